# Initial kernel scaffold; baseline (speedup 1.0000x reference)
import jax
import jax.numpy as jnp
from jax.experimental import pallas as pl


def kernel(v, xq, yq):
    raise NotImplementedError("write your pallas kernel here")


# speedup vs baseline: 1.5420x; 1.5420x over previous
import functools

import jax
import jax.numpy as jnp
from jax import lax
from jax.experimental import pallas as pl
from jax.experimental.pallas import tpu as pltpu
from jax.experimental.pallas import tpu_sc as plsc

_B, _C, _H, _W = 2, 96, 512, 512
_HW = _H * _W
_N = _B * _HW
_NC, _NS, _L = 2, 16, 16
_NW = _NC * _NS
_Q = _N // _NW
_K = 128


def _sc_body(vflat, xf, yf, out,
             xq_v, yq_v, i00_v, i01_v, i10_v, i11_v,
             r00, r01, r10, r11, out_v, sem):
    wid = lax.axis_index("s") * _NC + lax.axis_index("c")
    base = wid * _Q
    offs = (wid // (_NW // _B)) * _HW

    def chunk_body(g, carry):
        qb = base + g * _K
        pltpu.sync_copy(xf.at[pl.ds(qb, _K)], xq_v)
        pltpu.sync_copy(yf.at[pl.ds(qb, _K)], yq_v)
        for t in range(_K // _L):
            sl = pl.ds(t * _L, _L)
            xv = xq_v[sl]
            yv = yq_v[sl]
            x0 = xv.astype(jnp.int32)
            y0 = yv.astype(jnp.int32)
            i00 = y0 * _W + x0 + offs
            i00_v[sl] = i00
            i01_v[sl] = i00 + 1
            i10_v[sl] = i00 + _W
            i11_v[sl] = i00 + _W + 1
            xq_v[sl] = xv - x0.astype(jnp.float32)
            yq_v[sl] = yv - y0.astype(jnp.float32)
        c0 = pltpu.async_copy(vflat.at[i00_v], r00, sem)
        c1 = pltpu.async_copy(vflat.at[i01_v], r01, sem)
        c2 = pltpu.async_copy(vflat.at[i10_v], r10, sem)
        c3 = pltpu.async_copy(vflat.at[i11_v], r11, sem)
        c0.wait()
        c1.wait()
        c2.wait()
        c3.wait()

        def qbody(t, carry2):
            fx16 = xq_v[pl.ds(t * _L, _L)]
            fy16 = yq_v[pl.ds(t * _L, _L)]
            for u in range(_L):
                i = t * _L + u
                fx = fx16[u]
                fy = fy16[u]
                for j in range(_C // _L):
                    cs = pl.ds(j * _L, _L)
                    a = r00[i, cs]
                    b = r01[i, cs]
                    c = r10[i, cs]
                    d = r11[i, cs]
                    top = a + fx * (b - a)
                    bot = c + fx * (d - c)
                    out_v[i, cs] = top + fy * (bot - top)
            return carry2

        lax.fori_loop(0, _K // _L, qbody, 0)
        pltpu.sync_copy(out_v, out.at[pl.ds(qb, _K)])
        return carry

    lax.fori_loop(0, _Q // _K, chunk_body, 0)


_interp_sc = functools.partial(
    pl.kernel,
    out_type=jax.ShapeDtypeStruct((_N, _C), jnp.float32),
    mesh=plsc.VectorSubcoreMesh(core_axis_name="c", subcore_axis_name="s"),
    compiler_params=pltpu.CompilerParams(use_tc_tiling_on_sc=False),
    scratch_types=[
        pltpu.VMEM((_K,), jnp.float32),
        pltpu.VMEM((_K,), jnp.float32),
        pltpu.VMEM((_K,), jnp.int32),
        pltpu.VMEM((_K,), jnp.int32),
        pltpu.VMEM((_K,), jnp.int32),
        pltpu.VMEM((_K,), jnp.int32),
        pltpu.VMEM((_K, _C), jnp.float32),
        pltpu.VMEM((_K, _C), jnp.float32),
        pltpu.VMEM((_K, _C), jnp.float32),
        pltpu.VMEM((_K, _C), jnp.float32),
        pltpu.VMEM((_K, _C), jnp.float32),
        pltpu.SemaphoreType.DMA,
    ],
)(_sc_body)


def kernel(v, xq, yq):
    vflat = jnp.transpose(v, (0, 2, 3, 1)).reshape(_N, _C)
    out_flat = _interp_sc(vflat, xq.reshape(_N), yq.reshape(_N))
    return out_flat.reshape(_B, _H, _W, _C).transpose(0, 3, 1, 2)


# speedup vs baseline: 1.9737x; 1.2799x over previous
import functools

import jax
import jax.numpy as jnp
from jax import lax
from jax.experimental import pallas as pl
from jax.experimental.pallas import tpu as pltpu
from jax.experimental.pallas import tpu_sc as plsc

_B, _C, _H, _W = 2, 96, 512, 512
_HW = _H * _W
_N = _B * _HW
_NC, _NS, _L = 2, 16, 16
_NW = _NC * _NS
_Q = _N // _NW
_K = 128
_NCH = _Q // _K


def _sc_body(vflat, xf, yf, out, *refs):
    slots = (refs[0:12], refs[12:24])
    out_sem = refs[24]
    wid = lax.axis_index("s") * _NC + lax.axis_index("c")
    base = wid * _Q
    offs = (wid // (_NW // _B)) * _HW

    def prep_fire(g, sl):
        xq_v, yq_v, i00_v, i01_v, i10_v, i11_v, r00, r01, r10, r11, _, sem = sl
        qb = base + g * _K
        pltpu.sync_copy(xf.at[pl.ds(qb, _K)], xq_v)
        pltpu.sync_copy(yf.at[pl.ds(qb, _K)], yq_v)
        for t in range(_K // _L):
            sl16 = pl.ds(t * _L, _L)
            xv = xq_v[sl16]
            yv = yq_v[sl16]
            x0 = xv.astype(jnp.int32)
            y0 = yv.astype(jnp.int32)
            i00 = y0 * _W + x0 + offs
            i00_v[sl16] = i00
            i01_v[sl16] = i00 + 1
            i10_v[sl16] = i00 + _W
            i11_v[sl16] = i00 + _W + 1
            xq_v[sl16] = xv - x0.astype(jnp.float32)
            yq_v[sl16] = yv - y0.astype(jnp.float32)
        pltpu.async_copy(vflat.at[i00_v], r00, sem)
        pltpu.async_copy(vflat.at[i01_v], r01, sem)
        pltpu.async_copy(vflat.at[i10_v], r10, sem)
        pltpu.async_copy(vflat.at[i11_v], r11, sem)

    def blend_store(g, sl):
        xq_v, yq_v, i00_v, i01_v, i10_v, i11_v, r00, r01, r10, r11, out_v, sem = sl
        pltpu.make_async_copy(vflat.at[i00_v], r00, sem).wait()
        pltpu.make_async_copy(vflat.at[i01_v], r01, sem).wait()
        pltpu.make_async_copy(vflat.at[i10_v], r10, sem).wait()
        pltpu.make_async_copy(vflat.at[i11_v], r11, sem).wait()

        def qbody(t, carry2):
            fx16 = xq_v[pl.ds(t * _L, _L)]
            fy16 = yq_v[pl.ds(t * _L, _L)]
            for u in range(_L):
                i = t * _L + u
                fx = fx16[u]
                fy = fy16[u]
                for j in range(_C // _L):
                    cs = pl.ds(j * _L, _L)
                    a = r00[i, cs]
                    b = r01[i, cs]
                    c = r10[i, cs]
                    d = r11[i, cs]
                    top = a + fx * (b - a)
                    bot = c + fx * (d - c)
                    out_v[i, cs] = top + fy * (bot - top)
            return carry2

        lax.fori_loop(0, _K // _L, qbody, 0)
        qb = base + g * _K
        pltpu.async_copy(out_v, out.at[pl.ds(qb, _K)], out_sem)

    prep_fire(0, slots[0])

    def pair_body(p, carry):
        for par in range(2):
            g = p * 2 + par
            gn = g + 1

            @pl.when(gn < _NCH)
            def _():
                prep_fire(gn, slots[1 - par])

            @pl.when(p > 0)
            def _():
                pltpu.make_async_copy(
                    slots[par][10], out.at[pl.ds(base, _K)], out_sem).wait()

            blend_store(g, slots[par])
        return carry

    lax.fori_loop(0, _NCH // 2, pair_body, 0)
    pltpu.make_async_copy(slots[0][10], out.at[pl.ds(base, _K)], out_sem).wait()
    pltpu.make_async_copy(slots[1][10], out.at[pl.ds(base, _K)], out_sem).wait()


def _slot_types():
    return [
        pltpu.VMEM((_K,), jnp.float32),
        pltpu.VMEM((_K,), jnp.float32),
        pltpu.VMEM((_K,), jnp.int32),
        pltpu.VMEM((_K,), jnp.int32),
        pltpu.VMEM((_K,), jnp.int32),
        pltpu.VMEM((_K,), jnp.int32),
        pltpu.VMEM((_K, _C), jnp.float32),
        pltpu.VMEM((_K, _C), jnp.float32),
        pltpu.VMEM((_K, _C), jnp.float32),
        pltpu.VMEM((_K, _C), jnp.float32),
        pltpu.VMEM((_K, _C), jnp.float32),
        pltpu.SemaphoreType.DMA,
    ]


_interp_sc = functools.partial(
    pl.kernel,
    out_type=jax.ShapeDtypeStruct((_N, _C), jnp.float32),
    mesh=plsc.VectorSubcoreMesh(core_axis_name="c", subcore_axis_name="s"),
    compiler_params=pltpu.CompilerParams(use_tc_tiling_on_sc=False),
    scratch_types=_slot_types() + _slot_types() + [pltpu.SemaphoreType.DMA],
)(_sc_body)


def kernel(v, xq, yq):
    vflat = jnp.transpose(v, (0, 2, 3, 1)).reshape(_N, _C)
    out_flat = _interp_sc(vflat, xq.reshape(_N), yq.reshape(_N))
    return out_flat.reshape(_B, _H, _W, _C).transpose(0, 3, 1, 2)
